# baseline (speedup 1.0000x reference)
import jax
import jax.numpy as jnp
from jax.experimental import pallas as pl
from jax.experimental.pallas import tpu as pltpu

_NUM_HEADS = 8
_DIM_PER_HEAD = 32
_NUM_CLUSTERS = 512
_ROW_BLOCK = 512


def _vq_kernel(x_ref, w_ref, msq_ref, out_ref):
    xt = x_ref[...].T
    ids = []
    for h in range(_NUM_HEADS):
        xh_t = xt[h * _DIM_PER_HEAD:(h + 1) * _DIM_PER_HEAD, :]
        cross2 = jax.lax.dot_general(
            w_ref[h], xh_t,
            dimension_numbers=(((1,), (0,)), ((), ())),
            preferred_element_type=jnp.float32,
        )
        dists = cross2 + msq_ref[h][:, None]
        ids.append(jnp.argmin(dists, axis=0).astype(jnp.int32))
    out_ref[...] = jnp.stack(ids, axis=0).T


@jax.jit
def kernel(x, means):
    b, n, f = x.shape
    h, d, k = _NUM_HEADS, _DIM_PER_HEAD, _NUM_CLUSTERS
    rows = b * n
    x2 = x.reshape(rows, f)
    w = -2.0 * means
    m_sq = jnp.sum(means * means, axis=-1)

    grid = rows // _ROW_BLOCK
    out = pl.pallas_call(
        _vq_kernel,
        grid=(grid,),
        in_specs=[
            pl.BlockSpec((_ROW_BLOCK, f), lambda i: (i, 0)),
            pl.BlockSpec((h, k, d), lambda i: (0, 0, 0)),
            pl.BlockSpec((h, k), lambda i: (0, 0)),
        ],
        out_specs=pl.BlockSpec((_ROW_BLOCK, h), lambda i: (i, 0)),
        out_shape=jax.ShapeDtypeStruct((rows, h), jnp.int32),
        compiler_params=pltpu.CompilerParams(
            dimension_semantics=("parallel",),
        ),
    )(x2, w, m_sq)
    return out.reshape(b, n, h)


# speedup vs baseline: 1.2397x; 1.2397x over previous
import jax
import jax.numpy as jnp
from jax.experimental import pallas as pl
from jax.experimental.pallas import tpu as pltpu

_NUM_HEADS = 8
_DIM_PER_HEAD = 32
_NUM_CLUSTERS = 512
_ROW_BLOCK = 1024


def _vq_kernel(x_ref, w_ref, msq_ref, out_ref):
    xt = x_ref[...].T
    for h in range(_NUM_HEADS):
        xh_t = xt[h * _DIM_PER_HEAD:(h + 1) * _DIM_PER_HEAD, :]
        cross2 = jax.lax.dot_general(
            w_ref[h], xh_t,
            dimension_numbers=(((1,), (0,)), ((), ())),
            preferred_element_type=jnp.float32,
        )
        dists = cross2 + msq_ref[h][:, None]
        out_ref[h, :] = jnp.argmin(dists, axis=0).astype(jnp.int32)


@jax.jit
def kernel(x, means):
    b, n, f = x.shape
    h, d, k = _NUM_HEADS, _DIM_PER_HEAD, _NUM_CLUSTERS
    rows = b * n
    x2 = x.reshape(rows, f)
    w = -2.0 * means
    m_sq = jnp.sum(means * means, axis=-1)

    grid = rows // _ROW_BLOCK
    out = pl.pallas_call(
        _vq_kernel,
        grid=(grid,),
        in_specs=[
            pl.BlockSpec((_ROW_BLOCK, f), lambda i: (i, 0)),
            pl.BlockSpec((h, k, d), lambda i: (0, 0, 0)),
            pl.BlockSpec((h, k), lambda i: (0, 0)),
        ],
        out_specs=pl.BlockSpec((h, _ROW_BLOCK), lambda i: (0, i)),
        out_shape=jax.ShapeDtypeStruct((h, rows), jnp.int32),
        compiler_params=pltpu.CompilerParams(
            dimension_semantics=("parallel",),
        ),
    )(x2, w, m_sq)
    return out.T.reshape(b, n, h)
